# Initial kernel scaffold; baseline (speedup 1.0000x reference)
import jax
import jax.numpy as jnp
from jax.experimental import pallas as pl


def kernel(tokens_embed, spans_start, spans_width, m, k, Sm_W0, Sm_b0, Sm_Wout, Sm_bout, c2f_W, c2f_b):
    raise NotImplementedError("write your pallas kernel here")


# speedup vs baseline: 549.7035x; 549.7035x over previous
import functools

import jax
import jax.numpy as jnp
from jax import lax
from jax.experimental import pallas as pl
from jax.experimental.pallas import tpu as pltpu
from jax.experimental.pallas import tpu_sc as plsc

T_TOK = 4096
D = 256
N_SPANS = 20000
N_PAD = 20480
FFNN = 1024
M_SEL = 256
K_ANT = 64
NEG_INF = float("-inf")

_CH = 128
_BPW = N_PAD // 32


def _sc_gather_body(s_hbm, e_hbm, tok_hbm, ea_hbm, eb_hbm, idx_v, rows_v, sem):
    nc = plsc.get_sparse_core_info().num_cores
    wid = lax.axis_index("s") * nc + lax.axis_index("c")
    base = wid * _BPW
    for src, dst in ((s_hbm, ea_hbm), (e_hbm, eb_hbm)):
        for c in range(_BPW // _CH):
            off = base + c * _CH
            pltpu.sync_copy(src.at[pl.ds(off, _CH)], idx_v)
            pltpu.async_copy(tok_hbm.at[idx_v], rows_v, sem).wait()
            pltpu.sync_copy(rows_v, dst.at[pl.ds(off, _CH)])


def _gather_spans(starts_pad, ends_pad, tokens_embed):
    mesh = plsc.VectorSubcoreMesh(core_axis_name="c", subcore_axis_name="s")
    f = pl.kernel(
        _sc_gather_body,
        out_type=[jax.ShapeDtypeStruct((N_PAD, D), jnp.float32),
                  jax.ShapeDtypeStruct((N_PAD, D), jnp.float32)],
        mesh=mesh,
        scratch_types=[pltpu.VMEM((_CH,), jnp.int32),
                       pltpu.VMEM((_CH, D), jnp.float32),
                       pltpu.SemaphoreType.DMA],
    )
    return f(starts_pad, ends_pad, tokens_embed)


_BLK = 1024


def _scores_body(ea_ref, eb_ref, w0_ref, b0_ref, wout_ref, bout_ref, out_ref):
    emb = jnp.concatenate([ea_ref[:], eb_ref[:]], axis=1)
    h = jnp.maximum(
        jnp.dot(emb, w0_ref[:], preferred_element_type=jnp.float32)
        + b0_ref[:], 0.0)
    sc = jnp.dot(h, wout_ref[:], preferred_element_type=jnp.float32)
    out_ref[:] = (sc[:, 0] + bout_ref[0]).reshape(1, 1, _BLK)


def _scores(ea, eb, w0, b0, wout, bout):
    grid = (N_PAD // _BLK,)
    out = pl.pallas_call(
        _scores_body,
        grid=grid,
        in_specs=[
            pl.BlockSpec((_BLK, D), lambda j: (j, 0)),
            pl.BlockSpec((_BLK, D), lambda j: (j, 0)),
            pl.BlockSpec((2 * D, FFNN), lambda j: (0, 0)),
            pl.BlockSpec((1, FFNN), lambda j: (0, 0)),
            pl.BlockSpec((FFNN, 1), lambda j: (0, 0)),
            pl.BlockSpec(memory_space=pltpu.SMEM),
        ],
        out_specs=pl.BlockSpec((1, 1, _BLK), lambda j: (j, 0, 0)),
        out_shape=jax.ShapeDtypeStruct((N_PAD // _BLK, 1, _BLK), jnp.float32),
        compiler_params=pltpu.CompilerParams(
            vmem_limit_bytes=100 * 1024 * 1024),
    )(ea, eb, w0, b0, wout, bout)
    return out.reshape(N_PAD)[:N_SPANS]


def _nms_body(s_ref, e_ref, idx_ref, sc_ref, m_ref,
              ids_out, s_out, e_out, sc_out):
    m = m_ref[0]
    lane = lax.broadcasted_iota(jnp.int32, (1, M_SEL), 1)

    def put(vec, t, x, on):
        return jnp.where(on & (lane == t), x, vec)

    def cond1(c):
        i, t = c[0], c[1]
        return (i < N_SPANS) & (t < m)

    def body1(c):
        i, t, ts, te, pos, ids, os_, oe, osc = c
        s = s_ref[i]
        e = e_ref[i]
        cross = (((s < ts) & (e < te) & (e >= ts))
                 | ((s > ts) & (s <= te) & (e > te)))
        ok = jnp.logical_not(jnp.any(cross))
        ts = put(ts, t, s, ok)
        te = put(te, t, e, ok)
        pos = put(pos, t, i, ok)
        ids = put(ids, t, idx_ref[i], ok)
        os_ = put(os_, t, s, ok)
        oe = put(oe, t, e, ok)
        osc = put(osc, t, sc_ref[i], ok)
        return i + 1, t + ok.astype(jnp.int32), ts, te, pos, ids, os_, oe, osc

    zi = jnp.zeros((1, M_SEL), jnp.int32)
    zf = jnp.zeros((1, M_SEL), jnp.float32)
    c = (jnp.int32(0), jnp.int32(0), zi, zi, zi - 1, zi, zi, zi, zf)
    c = lax.while_loop(cond1, body1, c)
    _, t, ts, te, pos, ids, os_, oe, osc = c

    def cond2(c):
        j, t = c[0], c[1]
        return (j < N_SPANS) & (t < m)

    def body2(c):
        j, t, ids, os_, oe, osc = c
        take = jnp.logical_not(jnp.any(pos == j))
        ids = put(ids, t, idx_ref[j], take)
        os_ = put(os_, t, s_ref[j], take)
        oe = put(oe, t, e_ref[j], take)
        osc = put(osc, t, sc_ref[j], take)
        return j + 1, t + take.astype(jnp.int32), ids, os_, oe, osc

    c2 = lax.while_loop(cond2, body2,
                        (jnp.int32(0), t, ids, os_, oe, osc))
    _, _, ids, os_, oe, osc = c2
    ids_out[:] = ids
    s_out[:] = os_
    e_out[:] = oe
    sc_out[:] = osc


def _nms(sorted_s, sorted_e, sorted_idx, sorted_sc, m):
    smem = pl.BlockSpec(memory_space=pltpu.SMEM)
    oshape = jax.ShapeDtypeStruct((1, M_SEL), jnp.int32)
    return pl.pallas_call(
        _nms_body,
        in_specs=[smem] * 5,
        out_specs=[pl.BlockSpec((1, M_SEL), lambda: (0, 0))] * 4,
        out_shape=[oshape, oshape, oshape,
                   jax.ShapeDtypeStruct((1, M_SEL), jnp.float32)],
    )(sorted_s, sorted_e, sorted_idx, sorted_sc,
      jnp.full((1,), m, jnp.int32))


def _c2f_body(ea_ref, eb_ref, w_ref, cb_ref, sc_ref, k_ref,
              val_out, idx_out, off_out):
    emb = jnp.concatenate([ea_ref[:], eb_ref[:]], axis=1)
    src = (jnp.dot(emb, w_ref[:], preferred_element_type=jnp.float32)
           + cb_ref[:])
    g = lax.dot_general(src, emb, (((1,), (1,)), ((), ())),
                        preferred_element_type=jnp.float32)
    row = lax.broadcasted_iota(jnp.int32, (M_SEL, M_SEL), 0)
    col = lax.broadcasted_iota(jnp.int32, (M_SEL, M_SEL), 1)
    allv = jnp.where(col < row, g, NEG_INF)

    lane_k = lax.broadcasted_iota(jnp.int32, (M_SEL, K_ANT), 1)
    vals = jnp.zeros((M_SEL, K_ANT), jnp.float32)
    idxs = jnp.zeros((M_SEL, K_ANT), jnp.int32)
    used = jnp.zeros((M_SEL, M_SEL), jnp.bool_)
    for kk in range(50):
        cand = jnp.where(used, NEG_INF, allv)
        mx = jnp.max(cand, axis=1, keepdims=True)
        hit = (cand == mx) & jnp.logical_not(used)
        idxj = jnp.min(jnp.where(hit, col, 1 << 30), axis=1, keepdims=True)
        used = used | (col == idxj)
        vals = jnp.where(lane_k == kk, mx, vals)
        idxs = jnp.where(lane_k == kk, idxj, idxs)

    ts = sc_ref[:]
    vals = jnp.where(vals == NEG_INF, NEG_INF, vals + ts)
    vals = jnp.where(lane_k < k_ref[0], vals, NEG_INF)
    rowk = lax.broadcasted_iota(jnp.int32, (M_SEL, K_ANT), 0)
    val_out[:] = vals
    idx_out[:] = idxs
    off_out[:] = rowk - idxs


def _c2f_topk(emb_a, emb_b, w, cb, top_score_col, k):
    return pl.pallas_call(
        _c2f_body,
        in_specs=[pl.BlockSpec((M_SEL, D), lambda: (0, 0)),
                  pl.BlockSpec((M_SEL, D), lambda: (0, 0)),
                  pl.BlockSpec((2 * D, 2 * D), lambda: (0, 0)),
                  pl.BlockSpec((1, 2 * D), lambda: (0, 0)),
                  pl.BlockSpec((M_SEL, 1), lambda: (0, 0)),
                  pl.BlockSpec(memory_space=pltpu.SMEM)],
        out_shape=[
            jax.ShapeDtypeStruct((M_SEL, K_ANT), jnp.float32),
            jax.ShapeDtypeStruct((M_SEL, K_ANT), jnp.int32),
            jax.ShapeDtypeStruct((M_SEL, K_ANT), jnp.int32),
        ],
    )(emb_a, emb_b, w, cb, top_score_col,
      jnp.full((1,), k, jnp.int32))


def kernel(tokens_embed, spans_start, spans_width, m, k,
           Sm_W0, Sm_b0, Sm_Wout, Sm_bout, c2f_W, c2f_b):
    spans_start = spans_start.astype(jnp.int32)
    spans_width = spans_width.astype(jnp.int32)
    spans_end = spans_start + spans_width
    starts_pad = jnp.pad(spans_start, (0, N_PAD - N_SPANS))
    ends_pad = jnp.pad(spans_end, (0, N_PAD - N_SPANS))

    ea, eb = _gather_spans(starts_pad, ends_pad, tokens_embed)
    scores = _scores(ea, eb, Sm_W0, Sm_b0.reshape(1, FFNN), Sm_Wout,
                     Sm_bout)

    order = jnp.argsort(-scores).astype(jnp.int32)
    sorted_s = jnp.take(spans_start, order)
    sorted_e = jnp.take(spans_end, order)
    sorted_sc = jnp.take(scores, order)

    ids, sel_s, sel_e, sel_sc = _nms(sorted_s, sorted_e, order, sorted_sc, m)

    emb_a = jnp.take(tokens_embed, sel_s.reshape(-1), axis=0)
    emb_b = jnp.take(tokens_embed, sel_e.reshape(-1), axis=0)

    vals, idxs, offs = _c2f_topk(
        emb_a, emb_b, c2f_W, c2f_b.reshape(1, 2 * D),
        sel_sc.reshape(M_SEL, 1), k)
    return vals[:, :50], idxs[:, :50], offs[:, :50]


# speedup vs baseline: 589.8789x; 1.0731x over previous
import functools

import jax
import jax.numpy as jnp
from jax import lax
from jax.experimental import pallas as pl
from jax.experimental.pallas import tpu as pltpu
from jax.experimental.pallas import tpu_sc as plsc

T_TOK = 4096
D = 256
N_SPANS = 20000
N_PAD = 20480
FFNN = 1024
M_SEL = 256
K_ANT = 64
NEG_INF = float("-inf")

_CH = 128
_BPW = N_PAD // 32


_NBUF = 3


def _sc_gather_body(s_hbm, e_hbm, tok_hbm, ea_hbm, eb_hbm,
                    idxs_v, rows_v, gsem, wsem):
    nc = plsc.get_sparse_core_info().num_cores
    wid = lax.axis_index("s") * nc + lax.axis_index("c")
    base = wid * _BPW
    pltpu.sync_copy(s_hbm.at[pl.ds(base, _BPW)], idxs_v.at[0])
    pltpu.sync_copy(e_hbm.at[pl.ds(base, _BPW)], idxs_v.at[1])

    nchunk = _BPW // _CH
    njobs = 2 * nchunk

    def job(jn):
        half, c = divmod(jn, nchunk)
        return half, c, (ea_hbm if half == 0 else eb_hbm)

    gcop = {}
    wcop = {}
    for jn in range(njobs + 1):
        if jn < njobs:
            b = jn % _NBUF
            if jn >= _NBUF:
                wcop[b].wait()
            half, c, _ = job(jn)
            gcop[b] = pltpu.async_copy(
                tok_hbm.at[idxs_v.at[half, pl.ds(c * _CH, _CH)]],
                rows_v.at[b], gsem.at[b])
        if jn >= 1:
            bb = (jn - 1) % _NBUF
            gcop[bb].wait()
            _, c, dst = job(jn - 1)
            wcop[bb] = pltpu.async_copy(
                rows_v.at[bb], dst.at[pl.ds(base + c * _CH, _CH)],
                wsem.at[bb])
    for jn in range(njobs - _NBUF, njobs):
        wcop[jn % _NBUF].wait()


def _gather_spans(starts_pad, ends_pad, tokens_embed):
    mesh = plsc.VectorSubcoreMesh(core_axis_name="c", subcore_axis_name="s")
    f = pl.kernel(
        _sc_gather_body,
        out_type=[jax.ShapeDtypeStruct((N_PAD, D), jnp.float32),
                  jax.ShapeDtypeStruct((N_PAD, D), jnp.float32)],
        mesh=mesh,
        scratch_types=[pltpu.VMEM((2, _BPW), jnp.int32),
                       pltpu.VMEM((_NBUF, _CH, D), jnp.float32),
                       pltpu.SemaphoreType.DMA((_NBUF,)),
                       pltpu.SemaphoreType.DMA((_NBUF,))],
    )
    return f(starts_pad, ends_pad, tokens_embed)


_BLK = 1024


def _scores_body(ea_ref, eb_ref, w0_ref, b0_ref, wout_ref, bout_ref, out_ref):
    emb = jnp.concatenate([ea_ref[:], eb_ref[:]], axis=1)
    h = jnp.maximum(
        jnp.dot(emb, w0_ref[:], preferred_element_type=jnp.float32)
        + b0_ref[:], 0.0)
    sc = jnp.dot(h, wout_ref[:], preferred_element_type=jnp.float32)
    out_ref[:] = (sc[:, 0] + bout_ref[0]).reshape(1, 1, _BLK)


def _scores(ea, eb, w0, b0, wout, bout):
    grid = (N_PAD // _BLK,)
    out = pl.pallas_call(
        _scores_body,
        grid=grid,
        in_specs=[
            pl.BlockSpec((_BLK, D), lambda j: (j, 0)),
            pl.BlockSpec((_BLK, D), lambda j: (j, 0)),
            pl.BlockSpec((2 * D, FFNN), lambda j: (0, 0)),
            pl.BlockSpec((1, FFNN), lambda j: (0, 0)),
            pl.BlockSpec((FFNN, 1), lambda j: (0, 0)),
            pl.BlockSpec(memory_space=pltpu.SMEM),
        ],
        out_specs=pl.BlockSpec((1, 1, _BLK), lambda j: (j, 0, 0)),
        out_shape=jax.ShapeDtypeStruct((N_PAD // _BLK, 1, _BLK), jnp.float32),
        compiler_params=pltpu.CompilerParams(
            vmem_limit_bytes=100 * 1024 * 1024),
    )(ea, eb, w0, b0, wout, bout)
    return out.reshape(N_PAD)[:N_SPANS]


def _nms_body(s_ref, e_ref, idx_ref, sc_ref, m_ref,
              ids_out, s_out, e_out, sc_out):
    m = m_ref[0]
    lane = lax.broadcasted_iota(jnp.int32, (1, M_SEL), 1)

    def put(vec, t, x, on):
        return jnp.where(on & (lane == t), x, vec)

    def cond1(c):
        i, t = c[0], c[1]
        return (i < N_SPANS) & (t < m)

    def body1(c):
        i, t, ts, te, pos, ids, os_, oe, osc = c
        s = s_ref[i]
        e = e_ref[i]
        cross = (((s < ts) & (e < te) & (e >= ts))
                 | ((s > ts) & (s <= te) & (e > te)))
        ok = jnp.logical_not(jnp.any(cross))
        ts = put(ts, t, s, ok)
        te = put(te, t, e, ok)
        pos = put(pos, t, i, ok)
        ids = put(ids, t, idx_ref[i], ok)
        os_ = put(os_, t, s, ok)
        oe = put(oe, t, e, ok)
        osc = put(osc, t, sc_ref[i], ok)
        return i + 1, t + ok.astype(jnp.int32), ts, te, pos, ids, os_, oe, osc

    zi = jnp.zeros((1, M_SEL), jnp.int32)
    zf = jnp.zeros((1, M_SEL), jnp.float32)
    c = (jnp.int32(0), jnp.int32(0), zi, zi, zi - 1, zi, zi, zi, zf)
    c = lax.while_loop(cond1, body1, c)
    _, t, ts, te, pos, ids, os_, oe, osc = c

    def cond2(c):
        j, t = c[0], c[1]
        return (j < N_SPANS) & (t < m)

    def body2(c):
        j, t, ids, os_, oe, osc = c
        take = jnp.logical_not(jnp.any(pos == j))
        ids = put(ids, t, idx_ref[j], take)
        os_ = put(os_, t, s_ref[j], take)
        oe = put(oe, t, e_ref[j], take)
        osc = put(osc, t, sc_ref[j], take)
        return j + 1, t + take.astype(jnp.int32), ids, os_, oe, osc

    c2 = lax.while_loop(cond2, body2,
                        (jnp.int32(0), t, ids, os_, oe, osc))
    _, _, ids, os_, oe, osc = c2
    ids_out[:] = ids
    s_out[:] = os_
    e_out[:] = oe
    sc_out[:] = osc


def _nms(sorted_s, sorted_e, sorted_idx, sorted_sc, m):
    smem = pl.BlockSpec(memory_space=pltpu.SMEM)
    oshape = jax.ShapeDtypeStruct((1, M_SEL), jnp.int32)
    return pl.pallas_call(
        _nms_body,
        in_specs=[smem] * 5,
        out_specs=[pl.BlockSpec((1, M_SEL), lambda: (0, 0))] * 4,
        out_shape=[oshape, oshape, oshape,
                   jax.ShapeDtypeStruct((1, M_SEL), jnp.float32)],
    )(sorted_s, sorted_e, sorted_idx, sorted_sc,
      jnp.full((1,), m, jnp.int32))


def _c2f_body(ea_ref, eb_ref, w_ref, cb_ref, sc_ref, k_ref,
              val_out, idx_out, off_out):
    emb = jnp.concatenate([ea_ref[:], eb_ref[:]], axis=1)
    src = (jnp.dot(emb, w_ref[:], preferred_element_type=jnp.float32)
           + cb_ref[:])
    g = lax.dot_general(src, emb, (((1,), (1,)), ((), ())),
                        preferred_element_type=jnp.float32)
    row = lax.broadcasted_iota(jnp.int32, (M_SEL, M_SEL), 0)
    col = lax.broadcasted_iota(jnp.int32, (M_SEL, M_SEL), 1)
    allv = jnp.where(col < row, g, NEG_INF)

    lane_k = lax.broadcasted_iota(jnp.int32, (M_SEL, K_ANT), 1)
    vals = jnp.zeros((M_SEL, K_ANT), jnp.float32)
    idxs = jnp.zeros((M_SEL, K_ANT), jnp.int32)
    used = jnp.zeros((M_SEL, M_SEL), jnp.bool_)
    for kk in range(50):
        cand = jnp.where(used, NEG_INF, allv)
        mx = jnp.max(cand, axis=1, keepdims=True)
        hit = (cand == mx) & jnp.logical_not(used)
        idxj = jnp.min(jnp.where(hit, col, 1 << 30), axis=1, keepdims=True)
        used = used | (col == idxj)
        vals = jnp.where(lane_k == kk, mx, vals)
        idxs = jnp.where(lane_k == kk, idxj, idxs)

    ts = sc_ref[:]
    vals = jnp.where(vals == NEG_INF, NEG_INF, vals + ts)
    vals = jnp.where(lane_k < k_ref[0], vals, NEG_INF)
    rowk = lax.broadcasted_iota(jnp.int32, (M_SEL, K_ANT), 0)
    val_out[:] = vals
    idx_out[:] = idxs
    off_out[:] = rowk - idxs


def _c2f_topk(emb_a, emb_b, w, cb, top_score_col, k):
    return pl.pallas_call(
        _c2f_body,
        in_specs=[pl.BlockSpec((M_SEL, D), lambda: (0, 0)),
                  pl.BlockSpec((M_SEL, D), lambda: (0, 0)),
                  pl.BlockSpec((2 * D, 2 * D), lambda: (0, 0)),
                  pl.BlockSpec((1, 2 * D), lambda: (0, 0)),
                  pl.BlockSpec((M_SEL, 1), lambda: (0, 0)),
                  pl.BlockSpec(memory_space=pltpu.SMEM)],
        out_shape=[
            jax.ShapeDtypeStruct((M_SEL, K_ANT), jnp.float32),
            jax.ShapeDtypeStruct((M_SEL, K_ANT), jnp.int32),
            jax.ShapeDtypeStruct((M_SEL, K_ANT), jnp.int32),
        ],
    )(emb_a, emb_b, w, cb, top_score_col,
      jnp.full((1,), k, jnp.int32))


def kernel(tokens_embed, spans_start, spans_width, m, k,
           Sm_W0, Sm_b0, Sm_Wout, Sm_bout, c2f_W, c2f_b):
    spans_start = spans_start.astype(jnp.int32)
    spans_width = spans_width.astype(jnp.int32)
    spans_end = spans_start + spans_width
    starts_pad = jnp.pad(spans_start, (0, N_PAD - N_SPANS))
    ends_pad = jnp.pad(spans_end, (0, N_PAD - N_SPANS))

    ea, eb = _gather_spans(starts_pad, ends_pad, tokens_embed)
    scores = _scores(ea, eb, Sm_W0, Sm_b0.reshape(1, FFNN), Sm_Wout,
                     Sm_bout)

    order = jnp.argsort(-scores).astype(jnp.int32)
    sorted_s = jnp.take(spans_start, order)
    sorted_e = jnp.take(spans_end, order)
    sorted_sc = jnp.take(scores, order)

    ids, sel_s, sel_e, sel_sc = _nms(sorted_s, sorted_e, order, sorted_sc, m)

    emb_a = jnp.take(tokens_embed, sel_s.reshape(-1), axis=0)
    emb_b = jnp.take(tokens_embed, sel_e.reshape(-1), axis=0)

    vals, idxs, offs = _c2f_topk(
        emb_a, emb_b, c2f_W, c2f_b.reshape(1, 2 * D),
        sel_sc.reshape(M_SEL, 1), k)
    return vals[:, :50], idxs[:, :50], offs[:, :50]


# speedup vs baseline: 680.2586x; 1.1532x over previous
import jax
import jax.numpy as jnp
from jax import lax
from jax.experimental import pallas as pl
from jax.experimental.pallas import tpu as pltpu
from jax.experimental.pallas import tpu_sc as plsc

T_TOK = 4096
D = 256
N_SPANS = 20000
N_PAD = 20480
WMAX = 32
FFNN = 1024
M_SEL = 256
K_ANT = 64
NEG_INF = float("-inf")

_RB = 512
_CH = 128
_BPW = N_PAD // 32


def _tbl_body(cur_ref, nxt_ref, w0a_ref, w0b_ref, b0_ref, wout_ref, bout_ref,
              out_ref):
    cur = cur_ref[:]
    win = jnp.concatenate([cur, nxt_ref[:WMAX]], axis=0)
    a = jnp.dot(cur, w0a_ref[:], preferred_element_type=jnp.float32)
    b = jnp.dot(win, w0b_ref[:], preferred_element_type=jnp.float32)
    for w in range(WMAX):
        h = jnp.maximum(a + b[w:_RB + w] + b0_ref[:], 0.0)
        c = jnp.dot(h, wout_ref[:], preferred_element_type=jnp.float32)
        out_ref[:, w] = c[:, 0] + bout_ref[0]


def _score_table(tokens_embed, w0, b0, wout, bout):
    tokpad = jnp.concatenate(
        [tokens_embed, jnp.zeros((_RB, D), jnp.float32)], axis=0)
    return pl.pallas_call(
        _tbl_body,
        grid=(T_TOK // _RB,),
        in_specs=[
            pl.BlockSpec((_RB, D), lambda j: (j, 0)),
            pl.BlockSpec((_RB, D), lambda j: (j + 1, 0)),
            pl.BlockSpec((D, FFNN), lambda j: (0, 0)),
            pl.BlockSpec((D, FFNN), lambda j: (0, 0)),
            pl.BlockSpec((1, FFNN), lambda j: (0, 0)),
            pl.BlockSpec((FFNN, 1), lambda j: (0, 0)),
            pl.BlockSpec(memory_space=pltpu.SMEM),
        ],
        out_specs=pl.BlockSpec((_RB, WMAX), lambda j: (j, 0)),
        out_shape=jax.ShapeDtypeStruct((T_TOK, WMAX), jnp.float32),
        compiler_params=pltpu.CompilerParams(
            vmem_limit_bytes=50 * 1024 * 1024),
    )(tokpad, tokpad, w0[:D], w0[D:], b0.reshape(1, FFNN), wout, bout)


def _sc_score_body(s_hbm, w_hbm, tbl_hbm, out_hbm, sv, wv, fidx, ov, gsem):
    nc = plsc.get_sparse_core_info().num_cores
    wid = lax.axis_index("s") * nc + lax.axis_index("c")
    base = wid * _BPW
    pltpu.sync_copy(s_hbm.at[pl.ds(base, _BPW)], sv)
    pltpu.sync_copy(w_hbm.at[pl.ds(base, _BPW)], wv)
    for ii in range(_BPW // 16):
        sl = pl.ds(ii * 16, 16)
        fidx[sl] = sv[sl] * WMAX + wv[sl]
    cops = [
        pltpu.async_copy(
            tbl_hbm.at[fidx.at[pl.ds(c * _CH, _CH)]],
            ov.at[pl.ds(c * _CH, _CH)], gsem)
        for c in range(_BPW // _CH)
    ]
    for cp in cops:
        cp.wait()
    pltpu.sync_copy(ov, out_hbm.at[pl.ds(base, _BPW)])


def _sc_scores(starts_pad, widths_pad, tbl_flat):
    mesh = plsc.VectorSubcoreMesh(core_axis_name="c", subcore_axis_name="s")
    f = pl.kernel(
        _sc_score_body,
        out_type=jax.ShapeDtypeStruct((N_PAD,), jnp.float32),
        mesh=mesh,
        scratch_types=[pltpu.VMEM((_BPW,), jnp.int32),
                       pltpu.VMEM((_BPW,), jnp.int32),
                       pltpu.VMEM((_BPW,), jnp.int32),
                       pltpu.VMEM((_BPW,), jnp.float32),
                       pltpu.SemaphoreType.DMA],
    )
    return f(starts_pad, widths_pad, tbl_flat)


def _nms_body(s_ref, e_ref, idx_ref, sc_ref, m_ref,
              ids_out, s_out, e_out, sc_out):
    m = m_ref[0]
    lane = lax.broadcasted_iota(jnp.int32, (1, M_SEL), 1)

    def put(vec, t, x, on):
        return jnp.where(on & (lane == t), x, vec)

    def cond1(c):
        i, t = c[0], c[1]
        return (i < N_SPANS) & (t < m)

    def body1(c):
        i, t, ts, te, pos, ids, os_, oe, osc = c
        s = s_ref[i]
        e = e_ref[i]
        cross = (((s < ts) & (e < te) & (e >= ts))
                 | ((s > ts) & (s <= te) & (e > te)))
        ok = jnp.logical_not(jnp.any(cross))
        ts = put(ts, t, s, ok)
        te = put(te, t, e, ok)
        pos = put(pos, t, i, ok)
        ids = put(ids, t, idx_ref[i], ok)
        os_ = put(os_, t, s, ok)
        oe = put(oe, t, e, ok)
        osc = put(osc, t, sc_ref[i], ok)
        return i + 1, t + ok.astype(jnp.int32), ts, te, pos, ids, os_, oe, osc

    zi = jnp.zeros((1, M_SEL), jnp.int32)
    zf = jnp.zeros((1, M_SEL), jnp.float32)
    c = (jnp.int32(0), jnp.int32(0), zi, zi, zi - 1, zi, zi, zi, zf)
    c = lax.while_loop(cond1, body1, c)
    _, t, ts, te, pos, ids, os_, oe, osc = c

    def cond2(c):
        j, t = c[0], c[1]
        return (j < N_SPANS) & (t < m)

    def body2(c):
        j, t, ids, os_, oe, osc = c
        take = jnp.logical_not(jnp.any(pos == j))
        ids = put(ids, t, idx_ref[j], take)
        os_ = put(os_, t, s_ref[j], take)
        oe = put(oe, t, e_ref[j], take)
        osc = put(osc, t, sc_ref[j], take)
        return j + 1, t + take.astype(jnp.int32), ids, os_, oe, osc

    c2 = lax.while_loop(cond2, body2,
                        (jnp.int32(0), t, ids, os_, oe, osc))
    _, _, ids, os_, oe, osc = c2
    ids_out[:] = ids
    s_out[:] = os_
    e_out[:] = oe
    sc_out[:] = osc


def _nms(sorted_s, sorted_e, sorted_idx, sorted_sc, m):
    smem = pl.BlockSpec(memory_space=pltpu.SMEM)
    oshape = jax.ShapeDtypeStruct((1, M_SEL), jnp.int32)
    return pl.pallas_call(
        _nms_body,
        in_specs=[smem] * 5,
        out_specs=[pl.BlockSpec((1, M_SEL), lambda: (0, 0))] * 4,
        out_shape=[oshape, oshape, oshape,
                   jax.ShapeDtypeStruct((1, M_SEL), jnp.float32)],
    )(sorted_s, sorted_e, sorted_idx, sorted_sc,
      jnp.full((1,), m, jnp.int32))


def _c2f_body(ea_ref, eb_ref, w_ref, cb_ref, sc_ref, k_ref,
              val_out, idx_out, off_out):
    emb = jnp.concatenate([ea_ref[:], eb_ref[:]], axis=1)
    src = (jnp.dot(emb, w_ref[:], preferred_element_type=jnp.float32)
           + cb_ref[:])
    g = lax.dot_general(src, emb, (((1,), (1,)), ((), ())),
                        preferred_element_type=jnp.float32)
    row = lax.broadcasted_iota(jnp.int32, (M_SEL, M_SEL), 0)
    col = lax.broadcasted_iota(jnp.int32, (M_SEL, M_SEL), 1)
    allv = jnp.where(col < row, g, NEG_INF)

    lane_k = lax.broadcasted_iota(jnp.int32, (M_SEL, K_ANT), 1)
    vals = jnp.zeros((M_SEL, K_ANT), jnp.float32)
    idxs = jnp.zeros((M_SEL, K_ANT), jnp.int32)
    used = jnp.zeros((M_SEL, M_SEL), jnp.bool_)
    for kk in range(50):
        cand = jnp.where(used, NEG_INF, allv)
        mx = jnp.max(cand, axis=1, keepdims=True)
        hit = (cand == mx) & jnp.logical_not(used)
        idxj = jnp.min(jnp.where(hit, col, 1 << 30), axis=1, keepdims=True)
        used = used | (col == idxj)
        vals = jnp.where(lane_k == kk, mx, vals)
        idxs = jnp.where(lane_k == kk, idxj, idxs)

    ts = sc_ref[:]
    vals = jnp.where(vals == NEG_INF, NEG_INF, vals + ts)
    vals = jnp.where(lane_k < k_ref[0], vals, NEG_INF)
    rowk = lax.broadcasted_iota(jnp.int32, (M_SEL, K_ANT), 0)
    val_out[:] = vals
    idx_out[:] = idxs
    off_out[:] = rowk - idxs


def _c2f_topk(emb_a, emb_b, w, cb, top_score_col, k):
    return pl.pallas_call(
        _c2f_body,
        in_specs=[pl.BlockSpec((M_SEL, D), lambda: (0, 0)),
                  pl.BlockSpec((M_SEL, D), lambda: (0, 0)),
                  pl.BlockSpec((2 * D, 2 * D), lambda: (0, 0)),
                  pl.BlockSpec((1, 2 * D), lambda: (0, 0)),
                  pl.BlockSpec((M_SEL, 1), lambda: (0, 0)),
                  pl.BlockSpec(memory_space=pltpu.SMEM)],
        out_shape=[
            jax.ShapeDtypeStruct((M_SEL, K_ANT), jnp.float32),
            jax.ShapeDtypeStruct((M_SEL, K_ANT), jnp.int32),
            jax.ShapeDtypeStruct((M_SEL, K_ANT), jnp.int32),
        ],
    )(emb_a, emb_b, w, cb, top_score_col,
      jnp.full((1,), k, jnp.int32))


def kernel(tokens_embed, spans_start, spans_width, m, k,
           Sm_W0, Sm_b0, Sm_Wout, Sm_bout, c2f_W, c2f_b):
    spans_start = spans_start.astype(jnp.int32)
    spans_width = spans_width.astype(jnp.int32)
    spans_end = spans_start + spans_width
    starts_pad = jnp.pad(spans_start, (0, N_PAD - N_SPANS))
    widths_pad = jnp.pad(spans_width, (0, N_PAD - N_SPANS))

    tbl = _score_table(tokens_embed, Sm_W0, Sm_b0, Sm_Wout, Sm_bout)
    scores = _sc_scores(starts_pad, widths_pad,
                        tbl.reshape(T_TOK * WMAX))[:N_SPANS]

    iota = lax.iota(jnp.int32, N_SPANS)
    _, order, sorted_s, sorted_e, sorted_sc = lax.sort(
        (-scores, iota, spans_start, spans_end, scores),
        dimension=0, is_stable=True, num_keys=1)

    ids, sel_s, sel_e, sel_sc = _nms(sorted_s, sorted_e, order, sorted_sc, m)

    emb_a = jnp.take(tokens_embed, sel_s.reshape(-1), axis=0)
    emb_b = jnp.take(tokens_embed, sel_e.reshape(-1), axis=0)

    vals, idxs, offs = _c2f_topk(
        emb_a, emb_b, c2f_W, c2f_b.reshape(1, 2 * D),
        sel_sc.reshape(M_SEL, 1), k)
    return vals[:, :50], idxs[:, :50], offs[:, :50]
